# Initial kernel scaffold; baseline (speedup 1.0000x reference)
import jax
import jax.numpy as jnp
from jax.experimental import pallas as pl


def kernel(node_features, edge_index, edge_features, W_self1, b_self1, W_msg1, b_msg1, ln_g1, ln_b1, W_self2, b_self2, W_msg2, b_msg2, ln_g2, ln_b2):
    raise NotImplementedError("write your pallas kernel here")


# speedup vs baseline: 2.8968x; 2.8968x over previous
import functools

import jax
import jax.numpy as jnp
from jax import lax
from jax.experimental import pallas as pl
from jax.experimental.pallas import tpu as pltpu
from jax.experimental.pallas import tpu_sc as plsc

N_N = 10000
N_PAD = 10240
D = 128
DE = 16
DEA = 32
NC = 2
NS = 16
NW = NC * NS
CHUNK = 64
ROWS_SUB = N_PAD // NS


def _sc_aggregate(num_chunks):
    mesh = plsc.VectorSubcoreMesh(core_axis_name="c", subcore_axis_name="s")

    def body(x_hbm, src_hbm, dst_hbm, zx_hbm, out_s,
             src_c, dst_c, xbuf, acc_x, sem):
        c = lax.axis_index("c")
        s = lax.axis_index("s")
        w = s * NC + c
        base = s * ROWS_SUB

        pltpu.sync_copy(zx_hbm, acc_x.at[pl.ds(base, ROWS_SUB)])
        plsc.subcore_barrier()

        @pl.loop(0, num_chunks)
        def chunk_body(j):
            jj = w * num_chunks + j
            pltpu.sync_copy(src_hbm.at[jj], src_c)
            pltpu.sync_copy(dst_hbm.at[jj], dst_c)
            pltpu.async_copy(x_hbm.at[src_c], xbuf, sem).wait()
            pltpu.sync_copy(xbuf, acc_x.at[dst_c], add=True)

        plsc.subcore_barrier()
        pltpu.sync_copy(acc_x.at[pl.ds(base, ROWS_SUB)], out_s.at[c * NS + s])

    return functools.partial(
        pl.kernel,
        out_type=jax.ShapeDtypeStruct((NC * NS, ROWS_SUB, D), jnp.float32),
        mesh=mesh,
        scratch_types=[
            pltpu.VMEM((CHUNK,), jnp.int32),
            pltpu.VMEM((CHUNK,), jnp.int32),
            pltpu.VMEM((CHUNK, D), jnp.float32),
            pltpu.VMEM_SHARED((N_PAD, D), jnp.float32),
            pltpu.SemaphoreType.DMA,
        ])(body)


def _sc_edge_aggregate(num_chunks):
    mesh = plsc.VectorSubcoreMesh(core_axis_name="c", subcore_axis_name="s")

    def body(ef_hbm, dst_hbm, zx_hbm, out_e, dst_c, ebuf, wbuf, acc_e):
        c = lax.axis_index("c")
        s = lax.axis_index("s")
        w = s * NC + c
        base = s * ROWS_SUB

        pltpu.sync_copy(zx_hbm, acc_e.at[pl.ds(base, ROWS_SUB)])
        pltpu.sync_copy(zx_hbm.at[pl.ds(0, CHUNK)], wbuf)
        plsc.subcore_barrier()

        @pl.loop(0, num_chunks)
        def chunk_body(j):
            jj = w * num_chunks + j
            pltpu.sync_copy(dst_hbm.at[jj], dst_c)
            pltpu.sync_copy(ef_hbm.at[pl.ds(jj * CHUNK, CHUNK)], ebuf)
            for i in range(CHUNK):
                wbuf[i, pl.ds(0, 16)] = ebuf[i, pl.ds(0, 16)]
                wbuf[i, pl.ds(16, 16)] = ebuf[i, pl.ds(16, 16)]
            pltpu.sync_copy(wbuf, acc_e.at[dst_c], add=True)

        plsc.subcore_barrier()
        pltpu.sync_copy(acc_e.at[pl.ds(base, ROWS_SUB)], out_e.at[c * NS + s])

    return functools.partial(
        pl.kernel,
        out_type=jax.ShapeDtypeStruct((NC * NS, ROWS_SUB, D), jnp.float32),
        mesh=mesh,
        scratch_types=[
            pltpu.VMEM((CHUNK,), jnp.int32),
            pltpu.VMEM((CHUNK, DEA), jnp.float32),
            pltpu.VMEM((CHUNK, D), jnp.float32),
            pltpu.VMEM_SHARED((N_PAD, D), jnp.float32),
        ])(body)


BLK = 512
GRID = N_PAD // BLK


def _dense_layer(relu):
    def body(x_ref, s_ref, e_ref, ws_ref, bs_ref, wx_ref, we_ref,
             bm_ref, g_ref, b_ref, o_ref):
        x = x_ref[...]
        sm = s_ref[0] + s_ref[1]
        ea = e_ref[0] + e_ref[1]
        em = ea[:, :DE]
        deg = ea[:, DE]
        agg = jnp.dot(sm, wx_ref[...], preferred_element_type=jnp.float32)
        agg = agg + jnp.dot(em, we_ref[...], preferred_element_type=jnp.float32)
        agg = agg + deg[:, None] * bm_ref[...]
        agg = agg / jnp.maximum(deg, 1.0)[:, None]
        comb = jnp.dot(x, ws_ref[...], preferred_element_type=jnp.float32)
        comb = comb + bs_ref[...] + agg
        mu = jnp.mean(comb, axis=-1, keepdims=True)
        var = jnp.mean((comb - mu) ** 2, axis=-1, keepdims=True)
        out = (comb - mu) * lax.rsqrt(var + 1e-5) * g_ref[...] + b_ref[...]
        if relu:
            out = jnp.maximum(out, 0.0)
        o_ref[...] = out

    return pl.pallas_call(
        body,
        grid=(GRID,),
        in_specs=[
            pl.BlockSpec((BLK, D), lambda i: (i, 0)),
            pl.BlockSpec((NC, BLK, D), lambda i: (0, i, 0)),
            pl.BlockSpec((NC, BLK, D), lambda i: (0, i, 0)),
            pl.BlockSpec((D, D), lambda i: (0, 0)),
            pl.BlockSpec((1, D), lambda i: (0, 0)),
            pl.BlockSpec((D, D), lambda i: (0, 0)),
            pl.BlockSpec((DE, D), lambda i: (0, 0)),
            pl.BlockSpec((1, D), lambda i: (0, 0)),
            pl.BlockSpec((1, D), lambda i: (0, 0)),
            pl.BlockSpec((1, D), lambda i: (0, 0)),
        ],
        out_specs=pl.BlockSpec((BLK, D), lambda i: (i, 0)),
        out_shape=jax.ShapeDtypeStruct((N_PAD, D), jnp.float32),
    )


def kernel(node_features, edge_index, edge_features,
           W_self1, b_self1, W_msg1, b_msg1, ln_g1, ln_b1,
           W_self2, b_self2, W_msg2, b_msg2, ln_g2, ln_b2):
    n_edges = edge_index.shape[1]
    e_pad = -(-n_edges // (NW * CHUNK)) * (NW * CHUNK)
    num_chunks = e_pad // (NW * CHUNK)

    src = edge_index[0].astype(jnp.int32)
    dst = edge_index[1].astype(jnp.int32)
    pad = e_pad - n_edges
    src_p = jnp.concatenate(
        [src, jnp.zeros((pad,), jnp.int32)]).reshape(NW * num_chunks, CHUNK)
    dst_p = jnp.concatenate(
        [dst, jnp.full((pad,), N_N, jnp.int32)]).reshape(NW * num_chunks,
                                                         CHUNK)
    ef_p = jnp.concatenate(
        [jnp.concatenate(
            [edge_features,
             jnp.ones((n_edges, 1), jnp.float32),
             jnp.zeros((n_edges, DEA - DE - 1), jnp.float32)], axis=1),
         jnp.zeros((pad, DEA), jnp.float32)], axis=0)

    zx = jnp.zeros((ROWS_SUB, D), jnp.float32)
    x_pad = jnp.pad(node_features, ((0, N_PAD - N_N), (0, 0)))

    ep = _sc_edge_aggregate(num_chunks)(ef_p, dst_p, zx)
    ep = ep.reshape(NC, N_PAD, D)

    s1p = _sc_aggregate(num_chunks)(node_features, src_p, dst_p, zx)
    s1p = s1p.reshape(NC, N_PAD, D)

    h1 = _dense_layer(relu=True)(
        x_pad, s1p, ep,
        W_self1, b_self1.reshape(1, D), W_msg1[:D], W_msg1[D:],
        b_msg1.reshape(1, D), ln_g1.reshape(1, D), ln_b1.reshape(1, D))

    s2p = _sc_aggregate(num_chunks)(h1, src_p, dst_p, zx)
    s2p = s2p.reshape(NC, N_PAD, D)

    h2 = _dense_layer(relu=False)(
        h1, s2p, ep,
        W_self2, b_self2.reshape(1, D), W_msg2[:D], W_msg2[D:],
        b_msg2.reshape(1, D), ln_g2.reshape(1, D), ln_b2.reshape(1, D))
    return h2[:N_N]


# speedup vs baseline: 3.3562x; 1.1586x over previous
import functools

import jax
import jax.numpy as jnp
from jax import lax
from jax.experimental import pallas as pl
from jax.experimental.pallas import tpu as pltpu
from jax.experimental.pallas import tpu_sc as plsc

N_N = 10000
N_PAD = 10240
D = 128
DE = 16
DEA = 32
NC = 2
NS = 16
NW = NC * NS
CHUNK = 64
ROWS_SUB = N_PAD // NS


def _sc_aggregate(num_chunks):
    assert num_chunks % 2 == 0
    mesh = plsc.VectorSubcoreMesh(core_axis_name="c", subcore_axis_name="s")

    def body(x_hbm, src_hbm, dst_hbm, zx_hbm, out_s,
             src0, dst0, src1, dst1, xbuf0, xbuf1, acc_x, sem0, sem1):
        c = lax.axis_index("c")
        s = lax.axis_index("s")
        w = s * NC + c
        base = s * ROWS_SUB
        j0 = w * num_chunks

        pltpu.sync_copy(zx_hbm, acc_x.at[pl.ds(base, ROWS_SUB)])
        plsc.subcore_barrier()

        pltpu.sync_copy(src_hbm.at[j0], src0)
        pltpu.sync_copy(dst_hbm.at[j0], dst0)
        gather0 = pltpu.async_copy(x_hbm.at[src0], xbuf0, sem0)
        pltpu.sync_copy(src_hbm.at[j0 + 1], src1)
        pltpu.sync_copy(dst_hbm.at[j0 + 1], dst1)

        half = num_chunks // 2

        @pl.loop(0, half)
        def pair_body(t):
            a = 2 * t
            not_last = t < half - 1
            pltpu.async_copy(x_hbm.at[src1], xbuf1, sem1)
            pltpu.make_async_copy(x_hbm.at[src0], xbuf0, sem0).wait()
            pltpu.sync_copy(xbuf0, acc_x.at[dst0], add=True)

            @pl.when(not_last)
            def _():
                pltpu.sync_copy(src_hbm.at[j0 + a + 2], src0)
                pltpu.sync_copy(dst_hbm.at[j0 + a + 2], dst0)
                pltpu.async_copy(x_hbm.at[src0], xbuf0, sem0)

            pltpu.make_async_copy(x_hbm.at[src1], xbuf1, sem1).wait()
            pltpu.sync_copy(xbuf1, acc_x.at[dst1], add=True)

            @pl.when(not_last)
            def _():
                pltpu.sync_copy(src_hbm.at[j0 + a + 3], src1)
                pltpu.sync_copy(dst_hbm.at[j0 + a + 3], dst1)

        del gather0
        plsc.subcore_barrier()
        pltpu.sync_copy(acc_x.at[pl.ds(base, ROWS_SUB)], out_s.at[c * NS + s])

    return functools.partial(
        pl.kernel,
        out_type=jax.ShapeDtypeStruct((NC * NS, ROWS_SUB, D), jnp.float32),
        mesh=mesh,
        scratch_types=[
            pltpu.VMEM((CHUNK,), jnp.int32),
            pltpu.VMEM((CHUNK,), jnp.int32),
            pltpu.VMEM((CHUNK,), jnp.int32),
            pltpu.VMEM((CHUNK,), jnp.int32),
            pltpu.VMEM((CHUNK, D), jnp.float32),
            pltpu.VMEM((CHUNK, D), jnp.float32),
            pltpu.VMEM_SHARED((N_PAD, D), jnp.float32),
            pltpu.SemaphoreType.DMA,
            pltpu.SemaphoreType.DMA,
        ])(body)


def _sc_edge_aggregate(num_chunks):
    mesh = plsc.VectorSubcoreMesh(core_axis_name="c", subcore_axis_name="s")

    def body(ef_hbm, dst_hbm, zx_hbm, out_e, dst_c, ebuf, wbuf, acc_e):
        c = lax.axis_index("c")
        s = lax.axis_index("s")
        w = s * NC + c
        base = s * ROWS_SUB

        pltpu.sync_copy(zx_hbm, acc_e.at[pl.ds(base, ROWS_SUB)])
        pltpu.sync_copy(zx_hbm.at[pl.ds(0, CHUNK)], wbuf)
        plsc.subcore_barrier()

        @pl.loop(0, num_chunks)
        def chunk_body(j):
            jj = w * num_chunks + j
            pltpu.sync_copy(dst_hbm.at[jj], dst_c)
            pltpu.sync_copy(ef_hbm.at[pl.ds(jj * CHUNK, CHUNK)], ebuf)
            for i in range(CHUNK):
                wbuf[i, pl.ds(0, 16)] = ebuf[i, pl.ds(0, 16)]
                wbuf[i, pl.ds(16, 16)] = ebuf[i, pl.ds(16, 16)]
            pltpu.sync_copy(wbuf, acc_e.at[dst_c], add=True)

        plsc.subcore_barrier()
        pltpu.sync_copy(acc_e.at[pl.ds(base, ROWS_SUB)], out_e.at[c * NS + s])

    return functools.partial(
        pl.kernel,
        out_type=jax.ShapeDtypeStruct((NC * NS, ROWS_SUB, D), jnp.float32),
        mesh=mesh,
        scratch_types=[
            pltpu.VMEM((CHUNK,), jnp.int32),
            pltpu.VMEM((CHUNK, DEA), jnp.float32),
            pltpu.VMEM((CHUNK, D), jnp.float32),
            pltpu.VMEM_SHARED((N_PAD, D), jnp.float32),
        ])(body)


BLK = 512
GRID = N_PAD // BLK


def _dense_layer(relu):
    def body(x_ref, s_ref, e_ref, ws_ref, bs_ref, wx_ref, we_ref,
             bm_ref, g_ref, b_ref, o_ref):
        x = x_ref[...]
        sm = s_ref[0] + s_ref[1]
        ea = e_ref[0] + e_ref[1]
        em = ea[:, :DE]
        deg = ea[:, DE]
        agg = jnp.dot(sm, wx_ref[...], preferred_element_type=jnp.float32)
        agg = agg + jnp.dot(em, we_ref[...], preferred_element_type=jnp.float32)
        agg = agg + deg[:, None] * bm_ref[...]
        agg = agg / jnp.maximum(deg, 1.0)[:, None]
        comb = jnp.dot(x, ws_ref[...], preferred_element_type=jnp.float32)
        comb = comb + bs_ref[...] + agg
        mu = jnp.mean(comb, axis=-1, keepdims=True)
        var = jnp.mean((comb - mu) ** 2, axis=-1, keepdims=True)
        out = (comb - mu) * lax.rsqrt(var + 1e-5) * g_ref[...] + b_ref[...]
        if relu:
            out = jnp.maximum(out, 0.0)
        o_ref[...] = out

    return pl.pallas_call(
        body,
        grid=(GRID,),
        in_specs=[
            pl.BlockSpec((BLK, D), lambda i: (i, 0)),
            pl.BlockSpec((NC, BLK, D), lambda i: (0, i, 0)),
            pl.BlockSpec((NC, BLK, D), lambda i: (0, i, 0)),
            pl.BlockSpec((D, D), lambda i: (0, 0)),
            pl.BlockSpec((1, D), lambda i: (0, 0)),
            pl.BlockSpec((D, D), lambda i: (0, 0)),
            pl.BlockSpec((DE, D), lambda i: (0, 0)),
            pl.BlockSpec((1, D), lambda i: (0, 0)),
            pl.BlockSpec((1, D), lambda i: (0, 0)),
            pl.BlockSpec((1, D), lambda i: (0, 0)),
        ],
        out_specs=pl.BlockSpec((BLK, D), lambda i: (i, 0)),
        out_shape=jax.ShapeDtypeStruct((N_PAD, D), jnp.float32),
    )


def kernel(node_features, edge_index, edge_features,
           W_self1, b_self1, W_msg1, b_msg1, ln_g1, ln_b1,
           W_self2, b_self2, W_msg2, b_msg2, ln_g2, ln_b2):
    n_edges = edge_index.shape[1]
    e_pad = -(-n_edges // (NW * CHUNK * 2)) * (NW * CHUNK * 2)
    num_chunks = e_pad // (NW * CHUNK)

    src = edge_index[0].astype(jnp.int32)
    dst = edge_index[1].astype(jnp.int32)
    pad = e_pad - n_edges
    src_p = jnp.concatenate(
        [src, jnp.zeros((pad,), jnp.int32)]).reshape(NW * num_chunks, CHUNK)
    dst_p = jnp.concatenate(
        [dst, jnp.full((pad,), N_N, jnp.int32)]).reshape(NW * num_chunks,
                                                         CHUNK)
    ef_p = jnp.concatenate(
        [jnp.concatenate(
            [edge_features,
             jnp.ones((n_edges, 1), jnp.float32),
             jnp.zeros((n_edges, DEA - DE - 1), jnp.float32)], axis=1),
         jnp.zeros((pad, DEA), jnp.float32)], axis=0)

    zx = jnp.zeros((ROWS_SUB, D), jnp.float32)
    x_pad = jnp.pad(node_features, ((0, N_PAD - N_N), (0, 0)))

    ep = _sc_edge_aggregate(num_chunks)(ef_p, dst_p, zx)
    ep = ep.reshape(NC, N_PAD, D)

    s1p = _sc_aggregate(num_chunks)(node_features, src_p, dst_p, zx)
    s1p = s1p.reshape(NC, N_PAD, D)

    h1 = _dense_layer(relu=True)(
        x_pad, s1p, ep,
        W_self1, b_self1.reshape(1, D), W_msg1[:D], W_msg1[D:],
        b_msg1.reshape(1, D), ln_g1.reshape(1, D), ln_b1.reshape(1, D))

    s2p = _sc_aggregate(num_chunks)(h1, src_p, dst_p, zx)
    s2p = s2p.reshape(NC, N_PAD, D)

    h2 = _dense_layer(relu=False)(
        h1, s2p, ep,
        W_self2, b_self2.reshape(1, D), W_msg2[:D], W_msg2[D:],
        b_msg2.reshape(1, D), ln_g2.reshape(1, D), ln_b2.reshape(1, D))
    return h2[:N_N]
